# baseline (speedup 1.0000x reference)
import functools

import jax
import jax.numpy as jnp
from jax import lax
from jax.experimental import pallas as pl
from jax.experimental.pallas import tpu as pltpu
from jax.experimental.pallas import tpu_sc as plsc

N = 10000
NPAD = 10240
E = 320000
D = 128

NC = 2
NS = 16
NW = NC * NS
K = 80
NCH = 128
EPW = NCH * K
EPAD = NW * EPW
DEGSZ = 2 * NPAD
DPT = DEGSZ // NS
RPT = NPAD // NS


def _deg_body(idx_hbm, out_hbm, idx_v, ones_v, zer_v, deg_sh):
    c = lax.axis_index("c")
    s = lax.axis_index("s")
    w = c * NS + s

    def fill(i, _):
        zer_v[pl.ds(i * 16, 16)] = jnp.zeros((16,), jnp.float32)
        return 0
    lax.fori_loop(0, DPT // 16, fill, 0)
    for i in range(K // 16):
        ones_v[pl.ds(i * 16, 16)] = jnp.ones((16,), jnp.float32)
    pltpu.sync_copy(zer_v, deg_sh.at[pl.ds(s * DPT, DPT)])
    pltpu.sync_copy(idx_hbm.at[w], idx_v)
    plsc.subcore_barrier()

    def body(j, _):
        pltpu.sync_copy(ones_v, deg_sh.at[idx_v.at[j]], add=True)
        return 0
    lax.fori_loop(0, 2 * NCH, body, 0)
    plsc.subcore_barrier()
    pltpu.sync_copy(deg_sh.at[pl.ds(s * DPT, DPT)],
                    out_hbm.at[c, pl.ds(s * DPT, DPT)])


def _agg_body(tab_hbm, src_hbm, dst_hbm, out_hbm,
              srcv, dstv, buf_a, zer_v, agg_sh):
    c = lax.axis_index("c")
    s = lax.axis_index("s")
    w = c * NS + s

    def fill(i, _):
        for k in range(D // 16):
            zer_v[i, pl.ds(k * 16, 16)] = jnp.zeros((16,), jnp.float32)
        return 0
    lax.fori_loop(0, 8, fill, 0)

    def zcopy(k, _):
        pltpu.sync_copy(zer_v, agg_sh.at[pl.ds(s * RPT + k * 8, 8)])
        return 0
    lax.fori_loop(0, RPT // 8, zcopy, 0)
    pltpu.sync_copy(src_hbm.at[w], srcv)
    pltpu.sync_copy(dst_hbm.at[w], dstv)
    plsc.subcore_barrier()

    def body(j, _):
        pltpu.sync_copy(tab_hbm.at[srcv.at[j]], buf_a)
        pltpu.sync_copy(buf_a, agg_sh.at[dstv.at[j]], add=True)
        return 0
    lax.fori_loop(0, NCH, body, 0)
    plsc.subcore_barrier()
    pltpu.sync_copy(agg_sh.at[pl.ds(s * RPT, RPT)],
                    out_hbm.at[c, pl.ds(s * RPT, RPT)])


@functools.lru_cache(maxsize=None)
def _sc_kernels():
    mesh = plsc.VectorSubcoreMesh(
        core_axis_name="c", subcore_axis_name="s",
        num_cores=NC, num_subcores=NS)
    deg = pl.kernel(
        _deg_body,
        out_type=jax.ShapeDtypeStruct((NC, DEGSZ), jnp.float32),
        mesh=mesh,
        scratch_types=[
            pltpu.VMEM((2 * NCH, K), jnp.int32),
            pltpu.VMEM((K,), jnp.float32),
            pltpu.VMEM((DPT,), jnp.float32),
            pltpu.VMEM_SHARED((DEGSZ,), jnp.float32),
        ],
    )
    agg = pl.kernel(
        _agg_body,
        out_type=jax.ShapeDtypeStruct((NC, NPAD, D), jnp.float32),
        mesh=mesh,
        scratch_types=[
            pltpu.VMEM((NCH, K), jnp.int32),
            pltpu.VMEM((NCH, K), jnp.int32),
            pltpu.VMEM((K, D), jnp.float32),
            pltpu.VMEM((8, D), jnp.float32),
            pltpu.VMEM_SHARED((NPAD, D), jnp.float32),
        ],
    )
    return deg, agg


def _eye128():
    r = lax.broadcasted_iota(jnp.int32, (128, 128), 0)
    co = lax.broadcasted_iota(jnp.int32, (128, 128), 1)
    return (r == co).astype(jnp.float32)


def _norm_row(deg_ref):
    d = deg_ref[0, 0] + deg_ref[1, 0]
    return lax.rsqrt(jnp.clip(d, 1.0, None))


def _scale_body(degs_ref, feats_ref, out_ref):
    ns = _norm_row(degs_ref)
    out_ref[...] = jnp.dot(_eye128() * ns, feats_ref[...],
                           preferred_element_type=jnp.float32)


def _layer_body(degd_ref, degs_ref, agg_ref, w_ref, b_ref, out_ref):
    nd = _norm_row(degd_ref)
    agg = agg_ref[0] + agg_ref[1]
    h = jnp.dot(_eye128() * nd, agg, preferred_element_type=jnp.float32)
    h = jnp.dot(h, w_ref[...], preferred_element_type=jnp.float32) + b_ref[...]
    h = jnp.maximum(h, 0.0)
    ns = _norm_row(degs_ref)
    out_ref[...] = jnp.dot(_eye128() * ns, h, preferred_element_type=jnp.float32)


def _final_body(degd_ref, agg_ref, w_ref, b_ref, out_ref):
    b = pl.program_id(0)
    nd = _norm_row(degd_ref)
    agg = agg_ref[0] + agg_ref[1]
    h = jnp.dot(_eye128() * nd, agg, preferred_element_type=jnp.float32)
    h = jnp.dot(h, w_ref[...], preferred_element_type=jnp.float32) + b_ref[...]
    h = jnp.maximum(h, 0.0)
    rows = b * 128 + lax.broadcasted_iota(jnp.int32, (128, 128), 0)
    h = jnp.where(rows < N, h, 0.0)
    part = jnp.sum(h, axis=0, keepdims=True)

    @pl.when(b == 0)
    def _():
        out_ref[...] = jnp.zeros_like(out_ref)

    out_ref[...] += part

    @pl.when(b == NPAD // 128 - 1)
    def _():
        out_ref[...] = out_ref[...] * (1.0 / N)


_GRID = NPAD // 128

_deg_spec_src = pl.BlockSpec((2, 1, 1, 128), lambda b: (0, b, 0, 0))
_deg_spec_dst = pl.BlockSpec((2, 1, 1, 128), lambda b: (0, NPAD // 128 + b, 0, 0))
_row_spec = pl.BlockSpec((128, D), lambda b: (b, 0))
_agg_spec = pl.BlockSpec((2, 128, D), lambda b: (0, b, 0))
_w_spec = pl.BlockSpec((D, D), lambda b: (0, 0))
_b_spec = pl.BlockSpec((1, D), lambda b: (0, 0))

_scale_call = pl.pallas_call(
    _scale_body,
    grid=(_GRID,),
    in_specs=[_deg_spec_src, _row_spec],
    out_specs=_row_spec,
    out_shape=jax.ShapeDtypeStruct((NPAD, D), jnp.float32),
)

_layer_call = pl.pallas_call(
    _layer_body,
    grid=(_GRID,),
    in_specs=[_deg_spec_dst, _deg_spec_src, _agg_spec, _w_spec, _b_spec],
    out_specs=_row_spec,
    out_shape=jax.ShapeDtypeStruct((NPAD, D), jnp.float32),
)

_final_call = pl.pallas_call(
    _final_body,
    grid=(_GRID,),
    in_specs=[_deg_spec_dst, _agg_spec, _w_spec, _b_spec],
    out_specs=pl.BlockSpec((1, D), lambda b: (0, 0)),
    out_shape=jax.ShapeDtypeStruct((1, D), jnp.float32),
)


def kernel(feats, edge_index, W1, b1, W2, b2):
    pad = jnp.full((EPAD - E,), NPAD - 1, jnp.int32)
    src = jnp.concatenate([edge_index[0], pad])
    dst = jnp.concatenate([edge_index[1], pad])
    src_w = src.reshape(NW, NCH, K)
    dst_w = dst.reshape(NW, NCH, K)
    deg_idx = jnp.concatenate([src, dst + NPAD]).reshape(NW, 2 * NCH, K)

    _deg_kernel, _agg_kernel = _sc_kernels()
    degp = _deg_kernel(deg_idx)
    degp3 = degp.reshape(NC, 2 * NPAD // 128, 1, 128)

    feats_p = jnp.pad(feats, ((0, NPAD - N), (0, 0)))
    h0 = _scale_call(degp3, feats_p)

    agg1 = _agg_kernel(h0, src_w, dst_w)
    h1s = _layer_call(degp3, degp3, agg1, W1, b1.reshape(1, D))

    agg2 = _agg_kernel(h1s, src_w, dst_w)
    return _final_call(degp3, agg2, W2, b2.reshape(1, D))


# speedup vs baseline: 2.1170x; 2.1170x over previous
import functools

import jax
import jax.numpy as jnp
from jax import lax
from jax.experimental import pallas as pl
from jax.experimental.pallas import tpu as pltpu
from jax.experimental.pallas import tpu_sc as plsc

N = 10000
NPAD = 10240
E = 320000
D = 128

NC = 2
NS = 16
NW = NC * NS
K = 80
NCH = 128
EPW = NCH * K
EPAD = NW * EPW
DEGSZ = 2 * NPAD
DPT = DEGSZ // NS
RPT = NPAD // NS


def _deg_body(idx_hbm, out_hbm, idx_v, ones_v, zer_v, deg_sh):
    c = lax.axis_index("c")
    s = lax.axis_index("s")
    w = c * NS + s

    def fill(i, _):
        zer_v[pl.ds(i * 16, 16)] = jnp.zeros((16,), jnp.float32)
        return 0
    lax.fori_loop(0, DPT // 16, fill, 0)
    for i in range(K // 16):
        ones_v[pl.ds(i * 16, 16)] = jnp.ones((16,), jnp.float32)
    pltpu.sync_copy(zer_v, deg_sh.at[pl.ds(s * DPT, DPT)])
    pltpu.sync_copy(idx_hbm.at[w], idx_v)
    plsc.subcore_barrier()

    def body(j, _):
        pltpu.sync_copy(ones_v, deg_sh.at[idx_v.at[j]], add=True)
        return 0
    lax.fori_loop(0, 2 * NCH, body, 0)
    plsc.subcore_barrier()
    pltpu.sync_copy(deg_sh.at[pl.ds(s * DPT, DPT)],
                    out_hbm.at[c, pl.ds(s * DPT, DPT)])


def _agg_body(tab_hbm, src_hbm, dst_hbm, out_hbm,
              srcv, dstv, buf_a, zer_v, agg_sh):
    c = lax.axis_index("c")
    s = lax.axis_index("s")
    w = c * NS + s

    def fill(i, _):
        for k in range(D // 16):
            zer_v[i, pl.ds(k * 16, 16)] = jnp.zeros((16,), jnp.float32)
        return 0
    lax.fori_loop(0, 8, fill, 0)

    def zcopy(k, _):
        pltpu.sync_copy(zer_v, agg_sh.at[pl.ds(s * RPT + k * 8, 8)])
        return 0
    lax.fori_loop(0, RPT // 8, zcopy, 0)
    pltpu.sync_copy(src_hbm.at[w], srcv)
    pltpu.sync_copy(dst_hbm.at[w], dstv)
    plsc.subcore_barrier()

    def body(j, _):
        pltpu.sync_copy(tab_hbm.at[srcv.at[j]], buf_a)
        pltpu.sync_copy(buf_a, agg_sh.at[dstv.at[j]], add=True)
        return 0
    lax.fori_loop(0, NCH, body, 0)
    plsc.subcore_barrier()
    pltpu.sync_copy(agg_sh.at[pl.ds(s * RPT, RPT)],
                    out_hbm.at[c, pl.ds(s * RPT, RPT)])


@functools.lru_cache(maxsize=None)
def _sc_kernels():
    mesh = plsc.VectorSubcoreMesh(
        core_axis_name="c", subcore_axis_name="s",
        num_cores=NC, num_subcores=NS)
    deg = pl.kernel(
        _deg_body,
        out_type=jax.ShapeDtypeStruct((NC, DEGSZ), jnp.float32),
        mesh=mesh,
        scratch_types=[
            pltpu.VMEM((2 * NCH, K), jnp.int32),
            pltpu.VMEM((K,), jnp.float32),
            pltpu.VMEM((DPT,), jnp.float32),
            pltpu.VMEM_SHARED((DEGSZ,), jnp.float32),
        ],
    )
    agg = pl.kernel(
        _agg_body,
        out_type=jax.ShapeDtypeStruct((NC, NPAD, D), jnp.float32),
        mesh=mesh,
        scratch_types=[
            pltpu.VMEM((NCH, K), jnp.int32),
            pltpu.VMEM((NCH, K), jnp.int32),
            pltpu.VMEM((K, D), jnp.float32),
            pltpu.VMEM((8, D), jnp.float32),
            pltpu.VMEM_SHARED((NPAD, D), jnp.float32),
        ],
    )
    return deg, agg


def _eye128():
    r = lax.broadcasted_iota(jnp.int32, (128, 128), 0)
    co = lax.broadcasted_iota(jnp.int32, (128, 128), 1)
    return (r == co).astype(jnp.float32)


def _norm_row(deg_ref):
    d = deg_ref[0, 0] + deg_ref[1, 0]
    return lax.rsqrt(jnp.clip(d, 1.0, None))


def _scale_body(degs_ref, feats_ref, out_ref):
    ns = _norm_row(degs_ref)
    out_ref[...] = jnp.dot(_eye128() * ns, feats_ref[...],
                           preferred_element_type=jnp.float32)


def _layer_body(degd_ref, degs_ref, agg_ref, w_ref, b_ref, out_ref):
    nd = _norm_row(degd_ref)
    agg = agg_ref[0] + agg_ref[1]
    h = jnp.dot(_eye128() * nd, agg, preferred_element_type=jnp.float32)
    h = jnp.dot(h, w_ref[...], preferred_element_type=jnp.float32) + b_ref[...]
    h = jnp.maximum(h, 0.0)
    ns = _norm_row(degs_ref)
    out_ref[...] = jnp.dot(_eye128() * ns, h, preferred_element_type=jnp.float32)


def _final_body(degd_ref, agg_ref, w_ref, b_ref, out_ref):
    b = pl.program_id(0)
    nd = _norm_row(degd_ref)
    agg = agg_ref[0] + agg_ref[1]
    h = jnp.dot(_eye128() * nd, agg, preferred_element_type=jnp.float32)
    h = jnp.dot(h, w_ref[...], preferred_element_type=jnp.float32) + b_ref[...]
    h = jnp.maximum(h, 0.0)
    rows = b * 128 + lax.broadcasted_iota(jnp.int32, (128, 128), 0)
    h = jnp.where(rows < N, h, 0.0)
    part = jnp.sum(h, axis=0, keepdims=True)

    @pl.when(b == 0)
    def _():
        out_ref[...] = jnp.zeros_like(out_ref)

    out_ref[...] += part

    @pl.when(b == NPAD // 128 - 1)
    def _():
        out_ref[...] = out_ref[...] * (1.0 / N)


_GRID = NPAD // 128

_deg_spec_src = pl.BlockSpec((2, 1, 1, 128), lambda b: (0, b, 0, 0))
_deg_spec_dst = pl.BlockSpec((2, 1, 1, 128), lambda b: (0, NPAD // 128 + b, 0, 0))
_row_spec = pl.BlockSpec((128, D), lambda b: (b, 0))
_agg_spec = pl.BlockSpec((2, 128, D), lambda b: (0, b, 0))
_w_spec = pl.BlockSpec((D, D), lambda b: (0, 0))
_b_spec = pl.BlockSpec((1, D), lambda b: (0, 0))

_scale_call = pl.pallas_call(
    _scale_body,
    grid=(_GRID,),
    in_specs=[_deg_spec_src, _row_spec],
    out_specs=_row_spec,
    out_shape=jax.ShapeDtypeStruct((NPAD, D), jnp.float32),
)

_layer_call = pl.pallas_call(
    _layer_body,
    grid=(_GRID,),
    in_specs=[_deg_spec_dst, _deg_spec_src, _agg_spec, _w_spec, _b_spec],
    out_specs=_row_spec,
    out_shape=jax.ShapeDtypeStruct((NPAD, D), jnp.float32),
)

_final_call = pl.pallas_call(
    _final_body,
    grid=(_GRID,),
    in_specs=[_deg_spec_dst, _agg_spec, _w_spec, _b_spec],
    out_specs=pl.BlockSpec((1, D), lambda b: (0, 0)),
    out_shape=jax.ShapeDtypeStruct((1, D), jnp.float32),
)


def kernel(feats, edge_index, W1, b1, W2, b2):
    pad = N + (jnp.arange(EPAD - E, dtype=jnp.int32) % (NPAD - N))
    src = jnp.concatenate([edge_index[0], pad])
    dst = jnp.concatenate([edge_index[1], pad])
    src_w = src.reshape(NW, NCH, K)
    dst_w = dst.reshape(NW, NCH, K)
    deg_idx = jnp.concatenate([src, dst + NPAD]).reshape(NW, 2 * NCH, K)

    _deg_kernel, _agg_kernel = _sc_kernels()
    degp = _deg_kernel(deg_idx)
    degp3 = degp.reshape(NC, 2 * NPAD // 128, 1, 128)

    feats_p = jnp.pad(feats, ((0, NPAD - N), (0, 0)))
    h0 = _scale_call(degp3, feats_p)

    agg1 = _agg_kernel(h0, src_w, dst_w)
    h1s = _layer_call(degp3, degp3, agg1, W1, b1.reshape(1, D))

    agg2 = _agg_kernel(h1s, src_w, dst_w)
    return _final_call(degp3, agg2, W2, b2.reshape(1, D))


# speedup vs baseline: 2.9179x; 1.3783x over previous
import functools

import jax
import jax.numpy as jnp
from jax import lax
from jax.experimental import pallas as pl
from jax.experimental.pallas import tpu as pltpu
from jax.experimental.pallas import tpu_sc as plsc

N = 10000
NPAD = 10240
E = 320000
D = 128

NC = 2
NS = 16
NW = NC * NS
K = 80
NCH = 128
EPW = NCH * K
EPAD = NW * EPW
PH = (64, 64)
DEGSZ = 2 * NPAD
DPT = DEGSZ // NS
RPT = NPAD // NS


def _deg_body(idx_hbm, out_hbm, idx_v, ones_v, zer_v, deg_sh):
    c = lax.axis_index("c")
    s = lax.axis_index("s")
    w = c * NS + s

    def fill(i, _):
        zer_v[pl.ds(i * 16, 16)] = jnp.zeros((16,), jnp.float32)
        return 0
    lax.fori_loop(0, DPT // 16, fill, 0)
    for i in range(K // 16):
        ones_v[pl.ds(i * 16, 16)] = jnp.ones((16,), jnp.float32)
    pltpu.sync_copy(zer_v, deg_sh.at[pl.ds(s * DPT, DPT)])
    pltpu.sync_copy(idx_hbm.at[w], idx_v)
    plsc.subcore_barrier()

    def body(j, _):
        pltpu.sync_copy(ones_v, deg_sh.at[idx_v.at[j]], add=True)
        return 0
    lax.fori_loop(0, 2 * NCH, body, 0)
    plsc.subcore_barrier()
    pltpu.sync_copy(deg_sh.at[pl.ds(s * DPT, DPT)],
                    out_hbm.at[c, pl.ds(s * DPT, DPT)])


def _agg_body(tab_hbm, src_hbm, dst_hbm, out_hbm,
              srcv, dstv, buf_a, buf_b, zer_v, sem_a, sem_b, agg_sh):
    c = lax.axis_index("c")
    s = lax.axis_index("s")
    w = c * NS + s

    def fill(i, _):
        for k in range(D // 16):
            zer_v[i, pl.ds(k * 16, 16)] = jnp.zeros((16,), jnp.float32)
        return 0
    lax.fori_loop(0, 8, fill, 0)

    def zcopy(k, _):
        pltpu.sync_copy(zer_v, agg_sh.at[pl.ds(s * RPT + k * 8, 8)])
        return 0
    lax.fori_loop(0, RPT // 8, zcopy, 0)
    plsc.subcore_barrier()

    def phase(base, cnt):
        pltpu.sync_copy(src_hbm.at[w, pl.ds(base, cnt)], srcv.at[pl.ds(0, cnt)])
        pltpu.sync_copy(dst_hbm.at[w, pl.ds(base, cnt)], dstv.at[pl.ds(0, cnt)])
        pltpu.async_copy(tab_hbm.at[srcv.at[0]], buf_a, sem_a)

        def body(t, _):
            ja = 2 * t
            jb = 2 * t + 1
            pltpu.async_copy(tab_hbm.at[srcv.at[jb]], buf_b, sem_b)
            pltpu.make_async_copy(tab_hbm.at[srcv.at[ja]], buf_a, sem_a).wait()
            pltpu.sync_copy(buf_a, agg_sh.at[dstv.at[ja]], add=True)

            @pl.when(jb + 1 < cnt)
            def _():
                pltpu.async_copy(tab_hbm.at[srcv.at[jb + 1]], buf_a, sem_a)

            pltpu.make_async_copy(tab_hbm.at[srcv.at[jb]], buf_b, sem_b).wait()
            pltpu.sync_copy(buf_b, agg_sh.at[dstv.at[jb]], add=True)
            return 0
        lax.fori_loop(0, cnt // 2, body, 0)

    phase(0, PH[0])
    phase(PH[0], PH[1])
    plsc.subcore_barrier()
    pltpu.sync_copy(agg_sh.at[pl.ds(s * RPT, RPT)],
                    out_hbm.at[c, pl.ds(s * RPT, RPT)])


@functools.lru_cache(maxsize=None)
def _sc_kernels():
    mesh = plsc.VectorSubcoreMesh(
        core_axis_name="c", subcore_axis_name="s",
        num_cores=NC, num_subcores=NS)
    deg = pl.kernel(
        _deg_body,
        out_type=jax.ShapeDtypeStruct((NC, DEGSZ), jnp.float32),
        mesh=mesh,
        scratch_types=[
            pltpu.VMEM((2 * NCH, K), jnp.int32),
            pltpu.VMEM((K,), jnp.float32),
            pltpu.VMEM((DPT,), jnp.float32),
            pltpu.VMEM_SHARED((DEGSZ,), jnp.float32),
        ],
    )
    agg = pl.kernel(
        _agg_body,
        out_type=jax.ShapeDtypeStruct((NC, NPAD, D), jnp.float32),
        mesh=mesh,
        scratch_types=[
            pltpu.VMEM((PH[1], K), jnp.int32),
            pltpu.VMEM((PH[1], K), jnp.int32),
            pltpu.VMEM((K, D), jnp.float32),
            pltpu.VMEM((K, D), jnp.float32),
            pltpu.VMEM((8, D), jnp.float32),
            pltpu.SemaphoreType.DMA,
            pltpu.SemaphoreType.DMA,
            pltpu.VMEM_SHARED((NPAD, D), jnp.float32),
        ],
    )
    return deg, agg


def _eye128():
    r = lax.broadcasted_iota(jnp.int32, (128, 128), 0)
    co = lax.broadcasted_iota(jnp.int32, (128, 128), 1)
    return (r == co).astype(jnp.float32)


def _norm_row(deg_ref):
    d = deg_ref[0, 0] + deg_ref[1, 0]
    return lax.rsqrt(jnp.clip(d, 1.0, None))


def _scale_body(degs_ref, feats_ref, out_ref):
    ns = _norm_row(degs_ref)
    out_ref[...] = jnp.dot(_eye128() * ns, feats_ref[...],
                           preferred_element_type=jnp.float32)


def _layer_body(degd_ref, degs_ref, agg_ref, w_ref, b_ref, out_ref):
    nd = _norm_row(degd_ref)
    agg = agg_ref[0] + agg_ref[1]
    h = jnp.dot(_eye128() * nd, agg, preferred_element_type=jnp.float32)
    h = jnp.dot(h, w_ref[...], preferred_element_type=jnp.float32) + b_ref[...]
    h = jnp.maximum(h, 0.0)
    ns = _norm_row(degs_ref)
    out_ref[...] = jnp.dot(_eye128() * ns, h, preferred_element_type=jnp.float32)


def _final_body(degd_ref, agg_ref, w_ref, b_ref, out_ref):
    b = pl.program_id(0)
    nd = _norm_row(degd_ref)
    agg = agg_ref[0] + agg_ref[1]
    h = jnp.dot(_eye128() * nd, agg, preferred_element_type=jnp.float32)
    h = jnp.dot(h, w_ref[...], preferred_element_type=jnp.float32) + b_ref[...]
    h = jnp.maximum(h, 0.0)
    rows = b * 128 + lax.broadcasted_iota(jnp.int32, (128, 128), 0)
    h = jnp.where(rows < N, h, 0.0)
    part = jnp.sum(h, axis=0, keepdims=True)

    @pl.when(b == 0)
    def _():
        out_ref[...] = jnp.zeros_like(out_ref)

    out_ref[...] += part

    @pl.when(b == NPAD // 128 - 1)
    def _():
        out_ref[...] = out_ref[...] * (1.0 / N)


_GRID = NPAD // 128

_deg_spec_src = pl.BlockSpec((2, 1, 1, 128), lambda b: (0, b, 0, 0))
_deg_spec_dst = pl.BlockSpec((2, 1, 1, 128), lambda b: (0, NPAD // 128 + b, 0, 0))
_row_spec = pl.BlockSpec((128, D), lambda b: (b, 0))
_agg_spec = pl.BlockSpec((2, 128, D), lambda b: (0, b, 0))
_w_spec = pl.BlockSpec((D, D), lambda b: (0, 0))
_b_spec = pl.BlockSpec((1, D), lambda b: (0, 0))

_scale_call = pl.pallas_call(
    _scale_body,
    grid=(_GRID,),
    in_specs=[_deg_spec_src, _row_spec],
    out_specs=_row_spec,
    out_shape=jax.ShapeDtypeStruct((NPAD, D), jnp.float32),
)

_layer_call = pl.pallas_call(
    _layer_body,
    grid=(_GRID,),
    in_specs=[_deg_spec_dst, _deg_spec_src, _agg_spec, _w_spec, _b_spec],
    out_specs=_row_spec,
    out_shape=jax.ShapeDtypeStruct((NPAD, D), jnp.float32),
)

_final_call = pl.pallas_call(
    _final_body,
    grid=(_GRID,),
    in_specs=[_deg_spec_dst, _agg_spec, _w_spec, _b_spec],
    out_specs=pl.BlockSpec((1, D), lambda b: (0, 0)),
    out_shape=jax.ShapeDtypeStruct((1, D), jnp.float32),
)


def kernel(feats, edge_index, W1, b1, W2, b2):
    pad = N + (jnp.arange(EPAD - E, dtype=jnp.int32) % (NPAD - N))
    src = jnp.concatenate([edge_index[0], pad])
    dst = jnp.concatenate([edge_index[1], pad])
    src_w = src.reshape(NW, NCH, K)
    dst_w = dst.reshape(NW, NCH, K)
    deg_idx = jnp.concatenate([src, dst + NPAD]).reshape(NW, 2 * NCH, K)

    _deg_kernel, _agg_kernel = _sc_kernels()
    degp = _deg_kernel(deg_idx)
    degp3 = degp.reshape(NC, 2 * NPAD // 128, 1, 128)

    feats_p = jnp.pad(feats, ((0, NPAD - N), (0, 0)))
    h0 = _scale_call(degp3, feats_p)

    agg1 = _agg_kernel(h0, src_w, dst_w)
    h1s = _layer_call(degp3, degp3, agg1, W1, b1.reshape(1, D))

    agg2 = _agg_kernel(h1s, src_w, dst_w)
    return _final_call(degp3, agg2, W2, b2.reshape(1, D))


# speedup vs baseline: 3.8220x; 1.3099x over previous
import functools

import jax
import jax.numpy as jnp
from jax import lax
from jax.experimental import pallas as pl
from jax.experimental.pallas import tpu as pltpu
from jax.experimental.pallas import tpu_sc as plsc

N = 10000
NPAD = 10240
E = 320000
D = 128

NC = 2
NS = 16
NW = NC * NS
K = 80
NCH = 128
EPW = NCH * K
EPAD = NW * EPW
PH = (64, 64)
DEGSZ = 2 * NPAD
DPT = DEGSZ // NS
RPT = NPAD // NS


def _deg_body(idx_hbm, out_hbm, idx_v, ones_v, zer_v, deg_sh):
    c = lax.axis_index("c")
    s = lax.axis_index("s")
    w = c * NS + s

    def fill(i, _):
        zer_v[pl.ds(i * 16, 16)] = jnp.zeros((16,), jnp.float32)
        return 0
    lax.fori_loop(0, DPT // 16, fill, 0)
    for i in range(K // 16):
        ones_v[pl.ds(i * 16, 16)] = jnp.ones((16,), jnp.float32)
    pltpu.sync_copy(zer_v, deg_sh.at[pl.ds(s * DPT, DPT)])
    pltpu.sync_copy(idx_hbm.at[w], idx_v)
    plsc.subcore_barrier()

    def body(j, _):
        pltpu.sync_copy(ones_v, deg_sh.at[idx_v.at[j]], add=True)
        return 0
    lax.fori_loop(0, 2 * NCH, body, 0)
    plsc.subcore_barrier()
    pltpu.sync_copy(deg_sh.at[pl.ds(s * DPT, DPT)],
                    out_hbm.at[c, pl.ds(s * DPT, DPT)])


def _agg_body(tab_hbm, src_hbm, dst_hbm, out_hbm,
              srcv, dstv, buf_a, buf_b, zer_v, sem_a, sem_b, agg_sh):
    c = lax.axis_index("c")
    s = lax.axis_index("s")
    w = c * NS + s

    def fill(i, _):
        for k in range(D // 16):
            zer_v[i, pl.ds(k * 16, 16)] = jnp.zeros((16,), jnp.float32)
        return 0
    lax.fori_loop(0, 8, fill, 0)

    def zcopy(k, _):
        pltpu.sync_copy(zer_v, agg_sh.at[pl.ds(s * RPT + k * 8, 8)])
        return 0
    lax.fori_loop(0, RPT // 8, zcopy, 0)
    plsc.subcore_barrier()

    def phase(base, cnt):
        pltpu.sync_copy(src_hbm.at[w, pl.ds(base, cnt)], srcv.at[pl.ds(0, cnt)])
        pltpu.sync_copy(dst_hbm.at[w, pl.ds(base, cnt)], dstv.at[pl.ds(0, cnt)])
        pltpu.async_copy(tab_hbm.at[srcv.at[0]], buf_a, sem_a)

        def body(t, _):
            ja = 2 * t
            jb = 2 * t + 1
            pltpu.async_copy(tab_hbm.at[srcv.at[jb]], buf_b, sem_b)
            pltpu.make_async_copy(tab_hbm.at[srcv.at[ja]], buf_a, sem_a).wait()
            pltpu.sync_copy(buf_a, agg_sh.at[dstv.at[ja]], add=True)

            @pl.when(jb + 1 < cnt)
            def _():
                pltpu.async_copy(tab_hbm.at[srcv.at[jb + 1]], buf_a, sem_a)

            pltpu.make_async_copy(tab_hbm.at[srcv.at[jb]], buf_b, sem_b).wait()
            pltpu.sync_copy(buf_b, agg_sh.at[dstv.at[jb]], add=True)
            return 0
        lax.fori_loop(0, cnt // 2, body, 0)

    phase(0, PH[0])
    phase(PH[0], PH[1])
    plsc.subcore_barrier()
    pltpu.sync_copy(agg_sh.at[pl.ds(s * RPT, RPT)],
                    out_hbm.at[c, pl.ds(s * RPT, RPT)])


@functools.lru_cache(maxsize=None)
def _sc_kernels():
    mesh = plsc.VectorSubcoreMesh(
        core_axis_name="c", subcore_axis_name="s",
        num_cores=NC, num_subcores=NS)
    deg = pl.kernel(
        _deg_body,
        out_type=jax.ShapeDtypeStruct((NC, DEGSZ), jnp.float32),
        mesh=mesh,
        scratch_types=[
            pltpu.VMEM((2 * NCH, K), jnp.int32),
            pltpu.VMEM((K,), jnp.float32),
            pltpu.VMEM((DPT,), jnp.float32),
            pltpu.VMEM_SHARED((DEGSZ,), jnp.float32),
        ],
    )
    agg = pl.kernel(
        _agg_body,
        out_type=jax.ShapeDtypeStruct((NC, NPAD, D), jnp.float32),
        mesh=mesh,
        scratch_types=[
            pltpu.VMEM((PH[1], K), jnp.int32),
            pltpu.VMEM((PH[1], K), jnp.int32),
            pltpu.VMEM((K, D), jnp.float32),
            pltpu.VMEM((K, D), jnp.float32),
            pltpu.VMEM((8, D), jnp.float32),
            pltpu.SemaphoreType.DMA,
            pltpu.SemaphoreType.DMA,
            pltpu.VMEM_SHARED((NPAD, D), jnp.float32),
        ],
    )
    return deg, agg


BR = 1024
_GRID = NPAD // BR


def _norm_col(deg_ref):
    d = deg_ref[0] + deg_ref[1]
    return lax.rsqrt(jnp.clip(d, 1.0, None))


def _scale_body(degs_ref, feats_ref, out_ref):
    out_ref[...] = feats_ref[...] * _norm_col(degs_ref)


def _layer_body(degd_ref, degs_ref, agg_ref, w_ref, b_ref, out_ref):
    h = (agg_ref[0] + agg_ref[1]) * _norm_col(degd_ref)
    h = jnp.dot(h, w_ref[...], preferred_element_type=jnp.float32) + b_ref[...]
    h = jnp.maximum(h, 0.0)
    out_ref[...] = h * _norm_col(degs_ref)


def _final_body(degd_ref, agg_ref, w_ref, b_ref, out_ref):
    b = pl.program_id(0)
    h = (agg_ref[0] + agg_ref[1]) * _norm_col(degd_ref)
    h = jnp.dot(h, w_ref[...], preferred_element_type=jnp.float32) + b_ref[...]
    h = jnp.maximum(h, 0.0)
    rows = b * BR + lax.broadcasted_iota(jnp.int32, (BR, D), 0)
    h = jnp.where(rows < N, h, 0.0)
    part = jnp.sum(h, axis=0, keepdims=True)

    @pl.when(b == 0)
    def _():
        out_ref[...] = jnp.zeros_like(out_ref)

    out_ref[...] += part

    @pl.when(b == _GRID - 1)
    def _():
        out_ref[...] = out_ref[...] * (1.0 / N)


_degc_spec = pl.BlockSpec((2, BR, 1), lambda b: (0, b, 0))
_row_spec = pl.BlockSpec((BR, D), lambda b: (b, 0))
_agg_spec = pl.BlockSpec((2, BR, D), lambda b: (0, b, 0))
_w_spec = pl.BlockSpec((D, D), lambda b: (0, 0))
_b_spec = pl.BlockSpec((1, D), lambda b: (0, 0))

_scale_call = pl.pallas_call(
    _scale_body,
    grid=(_GRID,),
    in_specs=[_degc_spec, _row_spec],
    out_specs=_row_spec,
    out_shape=jax.ShapeDtypeStruct((NPAD, D), jnp.float32),
)

_layer_call = pl.pallas_call(
    _layer_body,
    grid=(_GRID,),
    in_specs=[_degc_spec, _degc_spec, _agg_spec, _w_spec, _b_spec],
    out_specs=_row_spec,
    out_shape=jax.ShapeDtypeStruct((NPAD, D), jnp.float32),
)

_final_call = pl.pallas_call(
    _final_body,
    grid=(_GRID,),
    in_specs=[_degc_spec, _agg_spec, _w_spec, _b_spec],
    out_specs=pl.BlockSpec((1, D), lambda b: (0, 0)),
    out_shape=jax.ShapeDtypeStruct((1, D), jnp.float32),
)


def kernel(feats, edge_index, W1, b1, W2, b2):
    pad = N + (jnp.arange(EPAD - E, dtype=jnp.int32) % (NPAD - N))
    src = jnp.concatenate([edge_index[0], pad])
    dst = jnp.concatenate([edge_index[1], pad])
    src_w = src.reshape(NW, NCH, K)
    dst_w = dst.reshape(NW, NCH, K)
    deg_idx = jnp.concatenate([src, dst + NPAD]).reshape(NW, 2 * NCH, K)

    _deg_kernel, _agg_kernel = _sc_kernels()
    degp = _deg_kernel(deg_idx)
    degs = degp[:, :NPAD, None]
    degd = degp[:, NPAD:, None]

    h0 = _scale_call(degs, feats if N == NPAD else jnp.pad(feats, ((0, NPAD - N), (0, 0))))

    agg1 = _agg_kernel(h0, src_w, dst_w)
    h1s = _layer_call(degd, degs, agg1, W1, b1.reshape(1, D))

    agg2 = _agg_kernel(h1s, src_w, dst_w)
    return _final_call(degd, agg2, W2, b2.reshape(1, D))


# speedup vs baseline: 3.9750x; 1.0400x over previous
import functools

import jax
import jax.numpy as jnp
from jax import lax
from jax.experimental import pallas as pl
from jax.experimental.pallas import tpu as pltpu
from jax.experimental.pallas import tpu_sc as plsc

N = 10000
NPAD = 10240
E = 320000
D = 128

NC = 2
NS = 16
NW = NC * NS
K = 80
NCH = 128
EPW = NCH * K
EPAD = NW * EPW
PH = (64, 64)
KD = 128
NDCH = 2 * EPW // KD
DEGSZ = 2 * NPAD
DPT = DEGSZ // NS
RPT = NPAD // NS


def _deg_body(idx_hbm, out_hbm, idx_v, ones_v, zer_v, deg_sh):
    c = lax.axis_index("c")
    s = lax.axis_index("s")
    w = c * NS + s

    def fill(i, _):
        zer_v[pl.ds(i * 16, 16)] = jnp.zeros((16,), jnp.float32)
        return 0
    lax.fori_loop(0, DPT // 16, fill, 0)
    for i in range(KD // 16):
        ones_v[pl.ds(i * 16, 16)] = jnp.ones((16,), jnp.float32)
    pltpu.sync_copy(zer_v, deg_sh.at[pl.ds(s * DPT, DPT)])
    pltpu.sync_copy(idx_hbm.at[w], idx_v)
    plsc.subcore_barrier()

    def body(j, _):
        pltpu.sync_copy(ones_v, deg_sh.at[idx_v.at[j]], add=True)
        return 0
    lax.fori_loop(0, NDCH, body, 0)
    plsc.subcore_barrier()
    pltpu.sync_copy(deg_sh.at[pl.ds(s * DPT, DPT)],
                    out_hbm.at[c, pl.ds(s * DPT, DPT)])


def _agg_body(tab_hbm, src_hbm, dst_hbm, out_hbm,
              srcv, dstv, buf_a, buf_b, zer_v, sem_a, sem_b, agg_sh):
    c = lax.axis_index("c")
    s = lax.axis_index("s")
    w = c * NS + s

    def fill(i, _):
        for k in range(D // 16):
            zer_v[i, pl.ds(k * 16, 16)] = jnp.zeros((16,), jnp.float32)
        return 0
    lax.fori_loop(0, 8, fill, 0)

    def zcopy(k, _):
        pltpu.sync_copy(zer_v, agg_sh.at[pl.ds(s * RPT + k * 8, 8)])
        return 0
    lax.fori_loop(0, RPT // 8, zcopy, 0)
    plsc.subcore_barrier()

    def phase(base, cnt):
        pltpu.sync_copy(src_hbm.at[w, pl.ds(base, cnt)], srcv.at[pl.ds(0, cnt)])
        pltpu.sync_copy(dst_hbm.at[w, pl.ds(base, cnt)], dstv.at[pl.ds(0, cnt)])
        pltpu.async_copy(tab_hbm.at[srcv.at[0]], buf_a, sem_a)

        def body(t, _):
            ja = 2 * t
            jb = 2 * t + 1
            pltpu.async_copy(tab_hbm.at[srcv.at[jb]], buf_b, sem_b)
            pltpu.make_async_copy(tab_hbm.at[srcv.at[ja]], buf_a, sem_a).wait()
            pltpu.sync_copy(buf_a, agg_sh.at[dstv.at[ja]], add=True)

            @pl.when(jb + 1 < cnt)
            def _():
                pltpu.async_copy(tab_hbm.at[srcv.at[jb + 1]], buf_a, sem_a)

            pltpu.make_async_copy(tab_hbm.at[srcv.at[jb]], buf_b, sem_b).wait()
            pltpu.sync_copy(buf_b, agg_sh.at[dstv.at[jb]], add=True)
            return 0
        lax.fori_loop(0, cnt // 2, body, 0)

    phase(0, PH[0])
    phase(PH[0], PH[1])
    plsc.subcore_barrier()
    pltpu.sync_copy(agg_sh.at[pl.ds(s * RPT, RPT)],
                    out_hbm.at[c, pl.ds(s * RPT, RPT)])


@functools.lru_cache(maxsize=None)
def _sc_kernels():
    mesh = plsc.VectorSubcoreMesh(
        core_axis_name="c", subcore_axis_name="s",
        num_cores=NC, num_subcores=NS)
    deg = pl.kernel(
        _deg_body,
        out_type=jax.ShapeDtypeStruct((NC, DEGSZ), jnp.float32),
        mesh=mesh,
        scratch_types=[
            pltpu.VMEM((NDCH, KD), jnp.int32),
            pltpu.VMEM((KD,), jnp.float32),
            pltpu.VMEM((DPT,), jnp.float32),
            pltpu.VMEM_SHARED((DEGSZ,), jnp.float32),
        ],
    )
    agg = pl.kernel(
        _agg_body,
        out_type=jax.ShapeDtypeStruct((NC, NPAD, D), jnp.float32),
        mesh=mesh,
        scratch_types=[
            pltpu.VMEM((PH[1], K), jnp.int32),
            pltpu.VMEM((PH[1], K), jnp.int32),
            pltpu.VMEM((K, D), jnp.float32),
            pltpu.VMEM((K, D), jnp.float32),
            pltpu.VMEM((8, D), jnp.float32),
            pltpu.SemaphoreType.DMA,
            pltpu.SemaphoreType.DMA,
            pltpu.VMEM_SHARED((NPAD, D), jnp.float32),
        ],
    )
    return deg, agg


BR = 2048
_GRID = NPAD // BR


def _norm_col(deg_ref):
    d = deg_ref[0] + deg_ref[1]
    return lax.rsqrt(jnp.clip(d, 1.0, None))


def _scale_body(degs_ref, feats_ref, out_ref):
    out_ref[...] = feats_ref[...] * _norm_col(degs_ref)


def _layer_body(degd_ref, degs_ref, agg_ref, w_ref, b_ref, out_ref):
    h = (agg_ref[0] + agg_ref[1]) * _norm_col(degd_ref)
    h = jnp.dot(h, w_ref[...], preferred_element_type=jnp.float32) + b_ref[...]
    h = jnp.maximum(h, 0.0)
    out_ref[...] = h * _norm_col(degs_ref)


def _final_body(degd_ref, agg_ref, w_ref, b_ref, out_ref):
    b = pl.program_id(0)
    h = (agg_ref[0] + agg_ref[1]) * _norm_col(degd_ref)
    h = jnp.dot(h, w_ref[...], preferred_element_type=jnp.float32) + b_ref[...]
    h = jnp.maximum(h, 0.0)
    rows = b * BR + lax.broadcasted_iota(jnp.int32, (BR, D), 0)
    h = jnp.where(rows < N, h, 0.0)
    part = jnp.sum(h, axis=0, keepdims=True)

    @pl.when(b == 0)
    def _():
        out_ref[...] = jnp.zeros_like(out_ref)

    out_ref[...] += part

    @pl.when(b == _GRID - 1)
    def _():
        out_ref[...] = out_ref[...] * (1.0 / N)


_degc_spec = pl.BlockSpec((2, BR, 1), lambda b: (0, b, 0))
_row_spec = pl.BlockSpec((BR, D), lambda b: (b, 0))
_agg_spec = pl.BlockSpec((2, BR, D), lambda b: (0, b, 0))
_w_spec = pl.BlockSpec((D, D), lambda b: (0, 0))
_b_spec = pl.BlockSpec((1, D), lambda b: (0, 0))

_scale_call = pl.pallas_call(
    _scale_body,
    grid=(_GRID,),
    in_specs=[_degc_spec, _row_spec],
    out_specs=_row_spec,
    out_shape=jax.ShapeDtypeStruct((NPAD, D), jnp.float32),
)

_layer_call = pl.pallas_call(
    _layer_body,
    grid=(_GRID,),
    in_specs=[_degc_spec, _degc_spec, _agg_spec, _w_spec, _b_spec],
    out_specs=_row_spec,
    out_shape=jax.ShapeDtypeStruct((NPAD, D), jnp.float32),
)

_final_call = pl.pallas_call(
    _final_body,
    grid=(_GRID,),
    in_specs=[_degc_spec, _agg_spec, _w_spec, _b_spec],
    out_specs=pl.BlockSpec((1, D), lambda b: (0, 0)),
    out_shape=jax.ShapeDtypeStruct((1, D), jnp.float32),
)


def kernel(feats, edge_index, W1, b1, W2, b2):
    pad = N + (jnp.arange(EPAD - E, dtype=jnp.int32) % (NPAD - N))
    src = jnp.concatenate([edge_index[0], pad])
    dst = jnp.concatenate([edge_index[1], pad])
    src_w = src.reshape(NW, NCH, K)
    dst_w = dst.reshape(NW, NCH, K)
    deg_idx = jnp.concatenate([src, dst + NPAD]).reshape(NW, NDCH, KD)

    _deg_kernel, _agg_kernel = _sc_kernels()
    degp = _deg_kernel(deg_idx)
    degs = degp[:, :NPAD, None]
    degd = degp[:, NPAD:, None]

    h0 = _scale_call(degs, feats if N == NPAD else jnp.pad(feats, ((0, NPAD - N), (0, 0))))

    agg1 = _agg_kernel(h0, src_w, dst_w)
    h1s = _layer_call(degd, degs, agg1, W1, b1.reshape(1, D))

    agg2 = _agg_kernel(h1s, src_w, dst_w)
    return _final_call(degd, agg2, W2, b2.reshape(1, D))


# speedup vs baseline: 4.0704x; 1.0240x over previous
import functools

import jax
import jax.numpy as jnp
from jax import lax
from jax.experimental import pallas as pl
from jax.experimental.pallas import tpu as pltpu
from jax.experimental.pallas import tpu_sc as plsc

N = 10000
NPAD = 10240
E = 320000
D = 128

NC = 2
NS = 16
NW = NC * NS
K = 80
NCH = 128
EPW = NCH * K
EPAD = NW * EPW
PH = (64, 64)
KD = 128
NDCH = 2 * EPW // KD
DEGSZ = 2 * NPAD
DPT = DEGSZ // NS
RPT = NPAD // NS


def _deg_body(idx_hbm, out_hbm, idx_v, ones_v, zer_v, deg_sh):
    c = lax.axis_index("c")
    s = lax.axis_index("s")
    w = c * NS + s

    def fill(i, _):
        zer_v[pl.ds(i * 16, 16)] = jnp.zeros((16,), jnp.float32)
        return 0
    lax.fori_loop(0, DPT // 16, fill, 0)
    for i in range(KD // 16):
        ones_v[pl.ds(i * 16, 16)] = jnp.ones((16,), jnp.float32)
    pltpu.sync_copy(zer_v, deg_sh.at[pl.ds(s * DPT, DPT)])
    pltpu.sync_copy(idx_hbm.at[w], idx_v)
    plsc.subcore_barrier()

    def body(j, _):
        pltpu.sync_copy(ones_v, deg_sh.at[idx_v.at[j]], add=True)
        return 0
    lax.fori_loop(0, NDCH, body, 0)
    plsc.subcore_barrier()
    pltpu.sync_copy(deg_sh.at[pl.ds(s * DPT, DPT)],
                    out_hbm.at[c, pl.ds(s * DPT, DPT)])


def _agg_body(tab_hbm, src_hbm, dst_hbm, out_hbm,
              srcv, dstv, buf_a, buf_b, zer_v, sem_a, sem_a2, sem_b, sem_b2,
              agg_sh):
    c = lax.axis_index("c")
    s = lax.axis_index("s")
    w = c * NS + s

    def fill(i, _):
        for k in range(D // 16):
            zer_v[i, pl.ds(k * 16, 16)] = jnp.zeros((16,), jnp.float32)
        return 0
    lax.fori_loop(0, 8, fill, 0)

    def zcopy(k, _):
        pltpu.sync_copy(zer_v, agg_sh.at[pl.ds(s * RPT + k * 8, 8)])
        return 0
    lax.fori_loop(0, RPT // 8, zcopy, 0)
    plsc.subcore_barrier()

    H = K // 2

    def gstart(j, buf, sx, sy):
        pltpu.async_copy(tab_hbm.at[srcv.at[j, pl.ds(0, H)]],
                         buf.at[pl.ds(0, H)], sx)
        pltpu.async_copy(tab_hbm.at[srcv.at[j, pl.ds(H, H)]],
                         buf.at[pl.ds(H, H)], sy)

    def gwait(j, buf, sx, sy):
        pltpu.make_async_copy(tab_hbm.at[srcv.at[j, pl.ds(0, H)]],
                              buf.at[pl.ds(0, H)], sx).wait()
        pltpu.make_async_copy(tab_hbm.at[srcv.at[j, pl.ds(H, H)]],
                              buf.at[pl.ds(H, H)], sy).wait()

    def phase(base, cnt):
        pltpu.sync_copy(src_hbm.at[w, pl.ds(base, cnt)], srcv.at[pl.ds(0, cnt)])
        pltpu.sync_copy(dst_hbm.at[w, pl.ds(base, cnt)], dstv.at[pl.ds(0, cnt)])
        gstart(0, buf_a, sem_a, sem_a2)

        def body(t, _):
            ja = 2 * t
            jb = 2 * t + 1
            gstart(jb, buf_b, sem_b, sem_b2)
            gwait(ja, buf_a, sem_a, sem_a2)
            pltpu.sync_copy(buf_a, agg_sh.at[dstv.at[ja]], add=True)

            @pl.when(jb + 1 < cnt)
            def _():
                gstart(jb + 1, buf_a, sem_a, sem_a2)

            gwait(jb, buf_b, sem_b, sem_b2)
            pltpu.sync_copy(buf_b, agg_sh.at[dstv.at[jb]], add=True)
            return 0
        lax.fori_loop(0, cnt // 2, body, 0)

    phase(0, PH[0])
    phase(PH[0], PH[1])
    plsc.subcore_barrier()
    pltpu.sync_copy(agg_sh.at[pl.ds(s * RPT, RPT)],
                    out_hbm.at[c, pl.ds(s * RPT, RPT)])


@functools.lru_cache(maxsize=None)
def _sc_kernels():
    mesh = plsc.VectorSubcoreMesh(
        core_axis_name="c", subcore_axis_name="s",
        num_cores=NC, num_subcores=NS)
    deg = pl.kernel(
        _deg_body,
        out_type=jax.ShapeDtypeStruct((NC, DEGSZ), jnp.float32),
        mesh=mesh,
        scratch_types=[
            pltpu.VMEM((NDCH, KD), jnp.int32),
            pltpu.VMEM((KD,), jnp.float32),
            pltpu.VMEM((DPT,), jnp.float32),
            pltpu.VMEM_SHARED((DEGSZ,), jnp.float32),
        ],
    )
    agg = pl.kernel(
        _agg_body,
        out_type=jax.ShapeDtypeStruct((NC, NPAD, D), jnp.float32),
        mesh=mesh,
        scratch_types=[
            pltpu.VMEM((PH[1], K), jnp.int32),
            pltpu.VMEM((PH[1], K), jnp.int32),
            pltpu.VMEM((K, D), jnp.float32),
            pltpu.VMEM((K, D), jnp.float32),
            pltpu.VMEM((8, D), jnp.float32),
            pltpu.SemaphoreType.DMA,
            pltpu.SemaphoreType.DMA,
            pltpu.SemaphoreType.DMA,
            pltpu.SemaphoreType.DMA,
            pltpu.VMEM_SHARED((NPAD, D), jnp.float32),
        ],
    )
    return deg, agg


BR = 2048
_GRID = NPAD // BR


def _norm_col(deg_ref):
    d = deg_ref[0] + deg_ref[1]
    return lax.rsqrt(jnp.clip(d, 1.0, None))


def _scale_body(degs_ref, feats_ref, out_ref):
    out_ref[...] = feats_ref[...] * _norm_col(degs_ref)


def _layer_body(degd_ref, degs_ref, agg_ref, w_ref, b_ref, out_ref):
    h = (agg_ref[0] + agg_ref[1]) * _norm_col(degd_ref)
    h = jnp.dot(h, w_ref[...], preferred_element_type=jnp.float32) + b_ref[...]
    h = jnp.maximum(h, 0.0)
    out_ref[...] = h * _norm_col(degs_ref)


def _final_body(degd_ref, agg_ref, w_ref, b_ref, out_ref):
    b = pl.program_id(0)
    h = (agg_ref[0] + agg_ref[1]) * _norm_col(degd_ref)
    h = jnp.dot(h, w_ref[...], preferred_element_type=jnp.float32) + b_ref[...]
    h = jnp.maximum(h, 0.0)
    rows = b * BR + lax.broadcasted_iota(jnp.int32, (BR, D), 0)
    h = jnp.where(rows < N, h, 0.0)
    part = jnp.sum(h, axis=0, keepdims=True)

    @pl.when(b == 0)
    def _():
        out_ref[...] = jnp.zeros_like(out_ref)

    out_ref[...] += part

    @pl.when(b == _GRID - 1)
    def _():
        out_ref[...] = out_ref[...] * (1.0 / N)


_degc_spec = pl.BlockSpec((2, BR, 1), lambda b: (0, b, 0))
_row_spec = pl.BlockSpec((BR, D), lambda b: (b, 0))
_agg_spec = pl.BlockSpec((2, BR, D), lambda b: (0, b, 0))
_w_spec = pl.BlockSpec((D, D), lambda b: (0, 0))
_b_spec = pl.BlockSpec((1, D), lambda b: (0, 0))

_scale_call = pl.pallas_call(
    _scale_body,
    grid=(_GRID,),
    in_specs=[_degc_spec, _row_spec],
    out_specs=_row_spec,
    out_shape=jax.ShapeDtypeStruct((NPAD, D), jnp.float32),
)

_layer_call = pl.pallas_call(
    _layer_body,
    grid=(_GRID,),
    in_specs=[_degc_spec, _degc_spec, _agg_spec, _w_spec, _b_spec],
    out_specs=_row_spec,
    out_shape=jax.ShapeDtypeStruct((NPAD, D), jnp.float32),
)

_final_call = pl.pallas_call(
    _final_body,
    grid=(_GRID,),
    in_specs=[_degc_spec, _agg_spec, _w_spec, _b_spec],
    out_specs=pl.BlockSpec((1, D), lambda b: (0, 0)),
    out_shape=jax.ShapeDtypeStruct((1, D), jnp.float32),
)


def kernel(feats, edge_index, W1, b1, W2, b2):
    pad = N + (jnp.arange(EPAD - E, dtype=jnp.int32) % (NPAD - N))
    src = jnp.concatenate([edge_index[0], pad])
    dst = jnp.concatenate([edge_index[1], pad])
    src_w = src.reshape(NW, NCH, K)
    dst_w = dst.reshape(NW, NCH, K)
    deg_idx = jnp.concatenate([src, dst + NPAD]).reshape(NW, NDCH, KD)

    _deg_kernel, _agg_kernel = _sc_kernels()
    degp = _deg_kernel(deg_idx)
    degs = degp[:, :NPAD, None]
    degd = degp[:, NPAD:, None]

    h0 = _scale_call(degs, feats if N == NPAD else jnp.pad(feats, ((0, NPAD - N), (0, 0))))

    agg1 = _agg_kernel(h0, src_w, dst_w)
    h1s = _layer_call(degd, degs, agg1, W1, b1.reshape(1, D))

    agg2 = _agg_kernel(h1s, src_w, dst_w)
    return _final_call(degd, agg2, W2, b2.reshape(1, D))
